# initial kernel scaffold (unmeasured)
import jax
import jax.numpy as jnp
from jax import lax
from jax.experimental import pallas as pl
from jax.experimental.pallas import tpu as pltpu


def kernel(x, A, B, C):
    Bb, S, D = x.shape
    N = A.shape[1]

    def body(x_ref, a_ref, b_ref, c_ref, out_ref,
             h_ref, comm_ref, send_sem, recv_sem, ack_sem):
        my_x = lax.axis_index("x")
        my_y = lax.axis_index("y")
        nbr = (my_x, 1 - my_y)

        barrier = pltpu.get_barrier_semaphore()
        pl.semaphore_signal(barrier, inc=1, device_id=nbr,
                            device_id_type=pl.DeviceIdType.MESH)
        pl.semaphore_wait(barrier, 1)

        aT = a_ref[...].T
        dAT = jnp.exp(aT)

        def step(t, h):
            xt = x_ref[:, pl.ds(t, 1), :]
            bt = b_ref[:, pl.ds(t, 1), :]
            ct = c_ref[:, pl.ds(t, 1), :]
            xt = xt.reshape(Bb, 1, D)
            btT = bt.reshape(Bb, N, 1)
            ctT = ct.reshape(Bb, N, 1)
            h = h * dAT[None] + xt * btT
            yt = jnp.sum(h * ctT, axis=1)
            out_ref[:, pl.ds(t, 1), :] = yt[:, None, :]
            return h

        h = lax.fori_loop(0, S, step, jnp.zeros((Bb, N, D), jnp.float32))
        h_ref[...] = h

        rdma = pltpu.make_async_remote_copy(
            src_ref=h_ref, dst_ref=comm_ref,
            send_sem=send_sem, recv_sem=recv_sem,
            device_id=nbr, device_id_type=pl.DeviceIdType.MESH,
        )

        @pl.when(my_y == 0)
        def _():
            rdma.start()
            rdma.wait_send()
            pl.semaphore_wait(ack_sem, 1)

        @pl.when(my_y == 1)
        def _():
            rdma.wait_recv()
            h0 = comm_ref[...]
            tpow = lax.broadcasted_iota(jnp.float32, (S, N, D), 0) + 1.0
            E = jnp.exp(aT[None, :, :] * tpow)
            for b in range(Bb):
                corr = jnp.sum(E * h0[b][None] * c_ref[b][:, :, None], axis=1)
                out_ref[b, :, :] = out_ref[b, :, :] + corr
            pl.semaphore_signal(ack_sem, inc=1, device_id=nbr,
                                device_id_type=pl.DeviceIdType.MESH)

    return pl.pallas_call(
        body,
        out_shape=jax.ShapeDtypeStruct((Bb, S, D), jnp.float32),
        in_specs=[pl.BlockSpec(memory_space=pltpu.VMEM)] * 4,
        out_specs=pl.BlockSpec(memory_space=pltpu.VMEM),
        scratch_shapes=[
            pltpu.VMEM((Bb, N, D), jnp.float32),
            pltpu.VMEM((Bb, N, D), jnp.float32),
            pltpu.SemaphoreType.DMA,
            pltpu.SemaphoreType.DMA,
            pltpu.SemaphoreType.REGULAR,
        ],
        compiler_params=pltpu.CompilerParams(collective_id=0),
    )(x, A, B, C)


# baseline (device time: 36365 ns/iter reference)
import jax
import jax.numpy as jnp
from jax import lax
from jax.experimental import pallas as pl
from jax.experimental.pallas import tpu as pltpu


def kernel(x, A, B, C):
    Bb, S, D = x.shape
    N = A.shape[1]

    def body(x_ref, a_ref, b_ref, c_ref, out_ref,
             h_ref, comm_ref, send_sem, recv_sem, ack_sem):
        my_x = lax.axis_index("x")
        my_y = lax.axis_index("y")
        nbr = (my_x, 1 - my_y)

        barrier = pltpu.get_barrier_semaphore()
        pl.semaphore_signal(barrier, inc=1, device_id=nbr,
                            device_id_type=pl.DeviceIdType.MESH)
        pl.semaphore_wait(barrier, 1)

        aT = a_ref[...].T
        dAT = jnp.exp(aT)

        def step(t, h):
            xt = x_ref[:, pl.ds(t, 1), :]
            bt = b_ref[:, pl.ds(t, 1), :]
            ct = c_ref[:, pl.ds(t, 1), :]
            xt = xt.reshape(Bb, 1, D)
            btT = bt.reshape(Bb, N, 1)
            ctT = ct.reshape(Bb, N, 1)
            h = h * dAT[None] + xt * btT
            yt = jnp.sum(h * ctT, axis=1)
            out_ref[:, pl.ds(t, 1), :] = yt[:, None, :]
            return h

        h = lax.fori_loop(0, S, step, jnp.zeros((Bb, N, D), jnp.float32))
        h_ref[...] = h

        rdma = pltpu.make_async_remote_copy(
            src_ref=h_ref, dst_ref=comm_ref,
            send_sem=send_sem, recv_sem=recv_sem,
            device_id=nbr, device_id_type=pl.DeviceIdType.MESH,
        )

        @pl.when(my_y == 0)
        def _():
            rdma.start()
            rdma.wait_send()
            pl.semaphore_wait(ack_sem, 1)

        @pl.when(my_y == 1)
        def _():
            rdma.wait_recv()
            h0 = comm_ref[...]
            tpow = lax.broadcasted_iota(jnp.int32, (S, N, D), 0).astype(jnp.float32) + 1.0
            E = jnp.exp(aT[None, :, :] * tpow)
            for b in range(Bb):
                corr = jnp.sum(E * h0[b][None] * c_ref[b][:, :, None], axis=1)
                out_ref[b, :, :] = out_ref[b, :, :] + corr
            pl.semaphore_signal(ack_sem, inc=1, device_id=nbr,
                                device_id_type=pl.DeviceIdType.MESH)

    return pl.pallas_call(
        body,
        out_shape=jax.ShapeDtypeStruct((Bb, S, D), jnp.float32),
        in_specs=[pl.BlockSpec(memory_space=pltpu.VMEM)] * 4,
        out_specs=pl.BlockSpec(memory_space=pltpu.VMEM),
        scratch_shapes=[
            pltpu.VMEM((Bb, N, D), jnp.float32),
            pltpu.VMEM((Bb, N, D), jnp.float32),
            pltpu.SemaphoreType.DMA,
            pltpu.SemaphoreType.DMA,
            pltpu.SemaphoreType.REGULAR,
        ],
        compiler_params=pltpu.CompilerParams(collective_id=0),
    )(x, A, B, C)


# device time: 19217 ns/iter; 1.8923x vs baseline; 1.8923x over previous
import jax
import jax.numpy as jnp
from jax import lax
from jax.experimental import pallas as pl
from jax.experimental.pallas import tpu as pltpu

K = 16
L = 16


def kernel(x, A, B, C):
    Bb, S, D = x.shape
    N = A.shape[1]
    assert S == K * L

    x2 = x.reshape(Bb, K, L, D)
    B2 = B.reshape(Bb, K, L, N)
    C2 = C.reshape(Bb, K, L, N)

    def body(x_ref, a_ref, b_ref, c_ref, out_ref,
             h_ref, comm_ref, send_sem, recv_sem, ack_sem):
        my_x = lax.axis_index("x")
        my_y = lax.axis_index("y")
        nbr = (my_x, 1 - my_y)

        barrier = pltpu.get_barrier_semaphore()
        pl.semaphore_signal(barrier, inc=1, device_id=nbr,
                            device_id_type=pl.DeviceIdType.MESH)
        pl.semaphore_wait(barrier, 1)

        @pl.when(my_y == 0)
        def _():
            comm_ref[...] = jnp.zeros((Bb, N, D), jnp.float32)

        aT = a_ref[...].T
        dAT = jnp.exp(aT)
        dAL = jnp.exp(aT * float(L))

        h = jnp.zeros((Bb, K, N, D), jnp.float32)
        for t in range(L):
            xt = x_ref[:, :, t, :].reshape(Bb, K, 1, D)
            bt = b_ref[:, :, t, :].reshape(Bb, K, N, 1)
            ct = c_ref[:, :, t, :].reshape(Bb, K, N, 1)
            h = h * dAT[None, None] + xt * bt
            out_ref[:, :, t, :] = jnp.sum(h * ct, axis=2)

        R_list = [jnp.zeros((Bb, N, D), jnp.float32)]
        for k in range(1, K + 1):
            R_list.append(R_list[k - 1] * dAL[None] + h[:, k - 1])
        h_ref[...] = R_list[K]
        R_local = jnp.stack(R_list[:K], axis=1)

        rdma = pltpu.make_async_remote_copy(
            src_ref=h_ref, dst_ref=comm_ref,
            send_sem=send_sem, recv_sem=recv_sem,
            device_id=nbr, device_id_type=pl.DeviceIdType.MESH,
        )

        @pl.when(my_y == 0)
        def _():
            rdma.start()

        @pl.when(my_y == 1)
        def _():
            rdma.wait_recv()

        h0 = comm_ref[...]
        dAkL_list = [jnp.ones((N, D), jnp.float32)]
        for k in range(1, K):
            dAkL_list.append(dAkL_list[k - 1] * dAL)
        dAkL = jnp.stack(dAkL_list, axis=0)
        R = R_local + h0[:, None] * dAkL[None]

        tpow = lax.broadcasted_iota(jnp.int32, (L, N, D), 0).astype(jnp.float32)
        EL = jnp.exp(aT[None] * (tpow + 1.0))
        for b in range(Bb):
            corr = jnp.sum(
                c_ref[b][:, :, :, None] * EL[None] * R[b][:, None], axis=2
            )
            out_ref[b] = out_ref[b] + corr

        @pl.when(my_y == 0)
        def _():
            rdma.wait_send()
            pl.semaphore_wait(ack_sem, 1)

        @pl.when(my_y == 1)
        def _():
            pl.semaphore_signal(ack_sem, inc=1, device_id=nbr,
                                device_id_type=pl.DeviceIdType.MESH)

    out = pl.pallas_call(
        body,
        out_shape=jax.ShapeDtypeStruct((Bb, K, L, D), jnp.float32),
        in_specs=[pl.BlockSpec(memory_space=pltpu.VMEM)] * 4,
        out_specs=pl.BlockSpec(memory_space=pltpu.VMEM),
        scratch_shapes=[
            pltpu.VMEM((Bb, N, D), jnp.float32),
            pltpu.VMEM((Bb, N, D), jnp.float32),
            pltpu.SemaphoreType.DMA,
            pltpu.SemaphoreType.DMA,
            pltpu.SemaphoreType.REGULAR,
        ],
        compiler_params=pltpu.CompilerParams(collective_id=0),
    )(x2, A, B2, C2)
    return out.reshape(Bb, S, D)
